# baseline (speedup 1.0000x reference)
import dataclasses
import functools

import jax
import jax.numpy as jnp
from jax import lax
from jax.experimental import pallas as pl
from jax.experimental.pallas import tpu as pltpu
from jax.experimental.pallas import tpu_sc as plsc

_C = 10
_LANES = 16
_RB = 8


def kernel(mask):
    B, _, H, W = mask.shape
    blk = _RB * W

    mesh = plsc.VectorSubcoreMesh(core_axis_name="core",
                                  subcore_axis_name="subcore")
    cp = pltpu.CompilerParams(use_tc_tiling_on_sc=True)
    if "needs_layout_passes" in pltpu.CompilerParams.__dataclass_fields__:
        cp = dataclasses.replace(cp, needs_layout_passes=False)

    @functools.partial(
        pl.kernel,
        out_type=jax.ShapeDtypeStruct((B, _C, H, W), jnp.int32),
        mesh=mesh,
        compiler_params=cp,
        scratch_types=[
            pltpu.SMEM((1,), jnp.int32),
            pltpu.VMEM((2 * blk,), jnp.int32),
        ],
    )
    def run(m_hbm, o_hbm, cnt_ref, save_ref):
        cnt_ref[0] = 0
        lanes = lax.iota(jnp.int32, _LANES)
        ones = jnp.ones((_LANES,), jnp.int32)
        zeros = jnp.zeros((_LANES,), jnp.int32)

        def body(m_vmem, o_vmem):
            it = cnt_ref[0]
            cnt_ref[0] = it + 1
            base = lax.rem(it, 2) * blk

            @pl.when(it < 2)
            def _():
                for c in range(_C):
                    @pl.loop(0, _RB)
                    def _(r):
                        @pl.loop(0, W, step=_LANES)
                        def _(j):
                            o_vmem[0, c, r, pl.ds(j, _LANES)] = zeros

            @pl.when(it >= 2)
            def _():
                @pl.loop(0, _RB)
                def _(r):
                    rvec = jnp.full((_LANES,), r, jnp.int32)

                    @pl.loop(0, W, step=_LANES)
                    def _(j):
                        v = save_ref[pl.ds(base + r * W + j, _LANES)]
                        plsc.store_scatter(
                            o_vmem, [zeros, v, rvec, lanes + j], zeros)

            @pl.loop(0, _RB)
            def _(r):
                rvec = jnp.full((_LANES,), r, jnp.int32)

                @pl.loop(0, W, step=_LANES)
                def _(j):
                    v = m_vmem[0, 0, r, pl.ds(j, _LANES)]
                    save_ref[pl.ds(base + r * W + j, _LANES)] = v
                    plsc.store_scatter(
                        o_vmem, [zeros, v, rvec, lanes + j], ones)

        pltpu.emit_pipeline(
            body,
            grid=(B, H // _RB),
            in_specs=[pl.BlockSpec((1, 1, _RB, W), lambda b, i: (b, 0, i, 0))],
            out_specs=[pl.BlockSpec((1, _C, _RB, W),
                                    lambda b, i: (b, 0, i, 0))],
            core_axis_name=("core", "subcore"),
            dimension_semantics=(pltpu.PARALLEL, pltpu.PARALLEL),
        )(m_hbm, o_hbm)

    return run(mask)


# speedup vs baseline: 1.6330x; 1.6330x over previous
import dataclasses
import functools

import jax
import jax.numpy as jnp
from jax.experimental import pallas as pl
from jax.experimental.pallas import tpu as pltpu
from jax.experimental.pallas import tpu_sc as plsc

_C = 10
_LANES = 16
_RB = 8
_WB = 256


def kernel(mask):
    B, _, H, W = mask.shape

    mesh = plsc.VectorSubcoreMesh(core_axis_name="core",
                                  subcore_axis_name="subcore")
    cp = pltpu.CompilerParams(use_tc_tiling_on_sc=True)
    if "needs_layout_passes" in pltpu.CompilerParams.__dataclass_fields__:
        cp = dataclasses.replace(cp, needs_layout_passes=False)

    @functools.partial(
        pl.kernel,
        out_type=jax.ShapeDtypeStruct((B, _C, H, W), jnp.int32),
        mesh=mesh,
        compiler_params=cp,
    )
    def run(m_hbm, o_hbm):
        def body(m_vmem, o_vmem):
            @pl.loop(0, _RB)
            def _(r):
                @pl.loop(0, _WB, step=_LANES, unroll=4)
                def _(j):
                    v = m_vmem[0, 0, r, pl.ds(j, _LANES)]
                    for c in range(_C):
                        o_vmem[0, c, r, pl.ds(j, _LANES)] = (
                            v == c).astype(jnp.int32)

        pltpu.emit_pipeline(
            body,
            grid=(B, H // _RB, W // _WB),
            in_specs=[pl.BlockSpec((1, 1, _RB, _WB),
                                   lambda b, i, k: (b, 0, i, k),
                                   pipeline_mode=pl.Buffered(buffer_count=4))],
            out_specs=[pl.BlockSpec((1, _C, _RB, _WB),
                                    lambda b, i, k: (b, 0, i, k),
                                    pipeline_mode=pl.Buffered(buffer_count=2))],
            core_axis_name=("core", "subcore"),
            dimension_semantics=(pltpu.PARALLEL, pltpu.PARALLEL,
                                 pltpu.PARALLEL),
        )(m_hbm, o_hbm)

    return run(mask)


# speedup vs baseline: 1.6621x; 1.0178x over previous
import dataclasses
import functools

import jax
import jax.numpy as jnp
from jax import lax
from jax.experimental import pallas as pl
from jax.experimental.pallas import tpu as pltpu
from jax.experimental.pallas import tpu_sc as plsc

_C = 10
_LANES = 16
_RB = 8
_WB = 256
_NB = 4


def kernel(mask):
    B, _, H, W = mask.shape
    tiles_h = H // _RB
    tiles_w = W // _WB
    tiles_per_img = tiles_h * tiles_w
    n_tiles = B * tiles_per_img
    n_workers = 32
    per_w = n_tiles // n_workers

    mesh = plsc.VectorSubcoreMesh(core_axis_name="core",
                                  subcore_axis_name="subcore")
    cp = pltpu.CompilerParams(use_tc_tiling_on_sc=True)
    if "needs_layout_passes" in pltpu.CompilerParams.__dataclass_fields__:
        cp = dataclasses.replace(cp, needs_layout_passes=False)

    @functools.partial(
        pl.kernel,
        out_type=jax.ShapeDtypeStruct((B, _C, H, W), jnp.int32),
        mesh=mesh,
        compiler_params=cp,
        scratch_types=[
            pltpu.VMEM((_NB, _RB, _WB), jnp.int32),
            pltpu.VMEM((_NB, _C, _RB, _WB), jnp.int32),
            pltpu.SemaphoreType.DMA((_NB,)),
            pltpu.SemaphoreType.DMA((_NB,)),
        ],
    )
    def run(m_hbm, o_hbm, in_v, out_v, in_sem, out_sem):
        wid = (lax.axis_index("subcore") * 2 + lax.axis_index("core"))
        t0 = wid * per_w

        def tile_coords(t):
            b = lax.div(t, tiles_per_img)
            rem = lax.rem(t, tiles_per_img)
            r0 = lax.div(rem, tiles_w) * _RB
            w0 = lax.rem(rem, tiles_w) * _WB
            return b, r0, w0

        def in_copy(t, s):
            b, r0, w0 = tile_coords(t)
            return pltpu.make_async_copy(
                m_hbm.at[b, 0, pl.ds(r0, _RB), pl.ds(w0, _WB)],
                in_v.at[s], in_sem.at[s])

        def out_copy(t, s):
            b, r0, w0 = tile_coords(t)
            return pltpu.make_async_copy(
                out_v.at[s],
                o_hbm.at[b, :, pl.ds(r0, _RB), pl.ds(w0, _WB)],
                out_sem.at[s])

        def compute(s):
            @pl.loop(0, _RB)
            def _(r):
                @pl.loop(0, _WB, step=_LANES)
                def _(j):
                    v = in_v[s, r, pl.ds(j, _LANES)]
                    for c in range(_C):
                        out_v[s, c, r, pl.ds(j, _LANES)] = (
                            v == c).astype(jnp.int32)

        for s in range(_NB):
            in_copy(t0 + s, s).start()

        for s in range(_NB):
            t = t0 + s
            in_copy(t, s).wait()
            compute(s)
            out_copy(t, s).start()
            in_copy(t + _NB, s).start()

        @pl.loop(_NB, per_w - _NB, step=_NB)
        def _(i0):
            for s in range(_NB):
                t = t0 + i0 + s
                in_copy(t, s).wait()
                out_copy(t - _NB, s).wait()
                compute(s)
                out_copy(t, s).start()
                in_copy(t + _NB, s).start()

        for s in range(_NB):
            t = t0 + per_w - _NB + s
            in_copy(t, s).wait()
            out_copy(t - _NB, s).wait()
            compute(s)
            out_copy(t, s).start()

        for s in range(_NB):
            out_copy(t0 + per_w - _NB + s, s).wait()

    return run(mask)


# speedup vs baseline: 1.6977x; 1.0214x over previous
import dataclasses
import functools

import jax
import jax.numpy as jnp
from jax.experimental import pallas as pl
from jax.experimental.pallas import tpu as pltpu
from jax.experimental.pallas import tpu_sc as plsc

_C = 10
_LANES = 16
_RB = 8


def kernel(mask):
    B, _, H, W = mask.shape

    mesh = plsc.VectorSubcoreMesh(core_axis_name="core",
                                  subcore_axis_name="subcore")
    cp = pltpu.CompilerParams(use_tc_tiling_on_sc=True)
    if "needs_layout_passes" in pltpu.CompilerParams.__dataclass_fields__:
        cp = dataclasses.replace(cp, needs_layout_passes=False)

    @functools.partial(
        pl.kernel,
        out_type=jax.ShapeDtypeStruct((B, _C, H, W), jnp.int32),
        mesh=mesh,
        compiler_params=cp,
    )
    def run(m_hbm, o_hbm):
        def body(m_vmem, o_vmem):
            @pl.loop(0, _RB)
            def _(r):
                @pl.loop(0, W, step=_LANES)
                def _(j):
                    v = m_vmem[0, 0, r, pl.ds(j, _LANES)]
                    for c in range(_C):
                        o_vmem[0, c, r, pl.ds(j, _LANES)] = (
                            v == c).astype(jnp.int32)

        pltpu.emit_pipeline(
            body,
            grid=(B, H // _RB),
            in_specs=[pl.BlockSpec((1, 1, _RB, W), lambda b, i: (b, 0, i, 0))],
            out_specs=[pl.BlockSpec((1, _C, _RB, W),
                                    lambda b, i: (b, 0, i, 0))],
            core_axis_name=("core", "subcore"),
            dimension_semantics=(pltpu.PARALLEL, pltpu.PARALLEL),
        )(m_hbm, o_hbm)

    return run(mask)
